# Initial kernel scaffold; baseline (speedup 1.0000x reference)
import jax
import jax.numpy as jnp
from jax.experimental import pallas as pl


def kernel(x, W1, b1, W2, b2, Wg, bg, Wd1, bd1, Wd2, bd2):
    raise NotImplementedError("write your pallas kernel here")


# speedup vs baseline: 1.4362x; 1.4362x over previous
import functools

import jax
import jax.numpy as jnp
from jax.experimental import pallas as pl
from jax.experimental.pallas import tpu as pltpu

D = 1024
E = 4
T_BLOCK = 256
MIN_K = 1.0
MAX_K = 4.0
TH_LO = 0.5
TH_HI = 2.0


def _routing_weights(xb, Wd1, bd1, Wd2, bd2, Wg, bg):
    hd = jax.nn.relu(jnp.dot(xb, Wd1, preferred_element_type=jnp.float32) + bd1)
    z = jnp.sum(hd * Wd2, axis=1, keepdims=True) + bd2
    ent = jax.nn.softplus(z)
    norm = jnp.clip((ent - TH_LO) / (TH_HI - TH_LO), 0.0, 1.0)
    k = jnp.clip(jnp.round(MIN_K + norm * (MAX_K - MIN_K)), MIN_K, MAX_K)
    logits = jnp.dot(xb, Wg, preferred_element_type=jnp.float32) + bg
    col = jax.lax.broadcasted_iota(jnp.int32, logits.shape, 1)
    rank = jnp.zeros_like(logits)
    for j in range(E):
        lj = logits[:, j:j + 1]
        rank = rank + jnp.where(
            (lj > logits) | ((lj == logits) & (col > j)), 1.0, 0.0)
    mask = rank < k
    m = jnp.max(logits, axis=1, keepdims=True)
    exps = jnp.where(mask, jnp.exp(logits - m), 0.0)
    return exps / jnp.sum(exps, axis=1, keepdims=True)


def _moe_kernel(x_ref, W1_ref, b1_ref, W2_ref, b2_ref, Wg_ref, bg_ref,
                Wd1_ref, bd1_ref, Wd2_ref, bd2_ref, out_ref, w_scr):
    e = pl.program_id(1)
    xb = x_ref[...]

    @pl.when(e == 0)
    def _():
        w_scr[...] = _routing_weights(
            xb, Wd1_ref[...], bd1_ref[...], Wd2_ref[...], bd2_ref[...],
            Wg_ref[...], bg_ref[...])

    wcol = jax.lax.broadcasted_iota(jnp.int32, (T_BLOCK, E), 1)
    w_e = jnp.sum(jnp.where(wcol == e, w_scr[...], 0.0), axis=1, keepdims=True)
    h = jax.nn.relu(jnp.dot(xb, W1_ref[0], preferred_element_type=jnp.float32)
                    + b1_ref[0])
    y = jnp.dot(h, W2_ref[0], preferred_element_type=jnp.float32) + b2_ref[0]
    contrib = w_e * y

    @pl.when(e == 0)
    def _():
        out_ref[...] = contrib

    @pl.when(e != 0)
    def _():
        out_ref[...] += contrib


def kernel(x, W1, b1, W2, b2, Wg, bg, Wd1, bd1, Wd2, bd2):
    B, N, Dm = x.shape
    T = B * N
    xf = x.reshape(T, Dm)
    n_blocks = T // T_BLOCK
    grid = (n_blocks, E)
    out = pl.pallas_call(
        _moe_kernel,
        grid=grid,
        in_specs=[
            pl.BlockSpec((T_BLOCK, Dm), lambda i, e: (i, 0)),
            pl.BlockSpec((1, Dm, 2 * Dm), lambda i, e: (e, 0, 0)),
            pl.BlockSpec((1, 1, 2 * Dm), lambda i, e: (e, 0, 0)),
            pl.BlockSpec((1, 2 * Dm, Dm), lambda i, e: (e, 0, 0)),
            pl.BlockSpec((1, 1, Dm), lambda i, e: (e, 0, 0)),
            pl.BlockSpec((Dm, E), lambda i, e: (0, 0)),
            pl.BlockSpec((1, E), lambda i, e: (0, 0)),
            pl.BlockSpec((Dm, Dm // 2), lambda i, e: (0, 0)),
            pl.BlockSpec((1, Dm // 2), lambda i, e: (0, 0)),
            pl.BlockSpec((1, Dm // 2), lambda i, e: (0, 0)),
            pl.BlockSpec((1, 1), lambda i, e: (0, 0)),
        ],
        out_specs=pl.BlockSpec((T_BLOCK, Dm), lambda i, e: (i, 0)),
        out_shape=jax.ShapeDtypeStruct((T, Dm), jnp.float32),
        scratch_shapes=[pltpu.VMEM((T_BLOCK, E), jnp.float32)],
        compiler_params=pltpu.CompilerParams(
            dimension_semantics=("arbitrary", "arbitrary"),
        ),
    )(xf, W1, b1.reshape(E, 1, 2 * Dm), W2, b2.reshape(E, 1, Dm),
      Wg, bg.reshape(1, E), Wd1, bd1.reshape(1, Dm // 2),
      Wd2.reshape(1, Dm // 2), bd2.reshape(1, 1))
    return out.reshape(B, N, Dm)


# speedup vs baseline: 2.1802x; 1.5180x over previous
import jax
import jax.numpy as jnp
from jax.experimental import pallas as pl
from jax.experimental.pallas import tpu as pltpu

D = 1024
E = 4
TB = 256
NBLK = 4096 // TB
MIN_K = 1.0
MAX_K = 4.0
TH_LO = 0.5
TH_HI = 2.0


def _routing(xb, Wd1, bd1, Wd2, bd2, Wg, bg):
    hd = jax.nn.relu(jnp.dot(xb, Wd1, preferred_element_type=jnp.float32) + bd1)
    z = jnp.dot(hd, Wd2, preferred_element_type=jnp.float32) + bd2
    ent = jax.nn.softplus(z)
    norm = jnp.clip((ent - TH_LO) / (TH_HI - TH_LO), 0.0, 1.0)
    k = jnp.clip(jnp.round(MIN_K + norm * (MAX_K - MIN_K)), MIN_K, MAX_K)
    logits = jnp.dot(xb, Wg, preferred_element_type=jnp.float32) + bg
    col = jax.lax.broadcasted_iota(jnp.int32, logits.shape, 1)
    rank = jnp.zeros_like(logits)
    for j in range(E):
        lj = logits[:, j:j + 1]
        rank = rank + jnp.where(
            (lj > logits) | ((lj == logits) & (col > j)), 1.0, 0.0)
    mask = rank < k
    m = jnp.max(logits, axis=1, keepdims=True)
    exps = jnp.where(mask, jnp.exp(logits - m), 0.0)
    w = exps / jnp.sum(exps, axis=1, keepdims=True)
    return w, mask


def _dispatch_kernel(x_ref, Wg_ref, bg_ref, Wd1_ref, bd1_ref, Wd2_ref, bd2_ref,
                     w_ref, lsl_ref, bs_ref, xg_ref,
                     gath_scr, carry_ref, sems):
    i = pl.program_id(0)

    @pl.when(i == 0)
    def _():
        for e in range(E):
            carry_ref[0, e] = 0

    xb = x_ref[...]
    w, mask = _routing(xb, Wd1_ref[...], bd1_ref[...], Wd2_ref[...],
                       bd2_ref[...], Wg_ref[...], bg_ref[...])
    maskf = mask.astype(jnp.float32)
    r0 = jax.lax.broadcasted_iota(jnp.int32, (TB, TB), 0)
    r1 = jax.lax.broadcasted_iota(jnp.int32, (TB, TB), 1)
    L = (r0 >= r1).astype(jnp.float32)
    cum = jnp.dot(L, maskf, preferred_element_type=jnp.float32)
    lsl = (cum - 1.0).astype(jnp.int32)

    w_ref[...] = w
    lsl_ref[...] = lsl

    col1 = jax.lax.broadcasted_iota(jnp.int32, (1, E), 1)
    vc = jnp.zeros((1, E), jnp.int32)
    for e in range(E):
        vc = jnp.where(col1 == e, carry_ref[0, e], vc)
    csum = jnp.sum(maskf, axis=0, keepdims=True).astype(jnp.int32)
    vc_new = vc + ((csum + 7) // 8) * 8
    riota = jax.lax.broadcasted_iota(jnp.int32, (NBLK + 1, E), 0)
    bs = jnp.where(riota == i, vc, bs_ref[...])
    bs_ref[...] = jnp.where(riota == NBLK, vc_new, bs)

    siota = jax.lax.broadcasted_iota(jnp.int32, (TB, TB), 1)
    for e in range(E):
        ohT = jnp.where((lsl[:, e:e + 1] == siota) & mask[:, e:e + 1],
                        1.0, 0.0)
        gath = jax.lax.dot_general(ohT, xb, (((0,), (0,)), ((), ())),
                                   preferred_element_type=jnp.float32)
        gath_scr[e] = gath
        start = pl.multiple_of(carry_ref[0, e], 8)
        cp = pltpu.make_async_copy(
            gath_scr.at[e], xg_ref.at[e, pl.ds(start, TB), :], sems.at[e])
        cp.start()
    for e in range(E):
        start = pl.multiple_of(carry_ref[0, e], 8)
        pltpu.make_async_copy(
            gath_scr.at[e], xg_ref.at[e, pl.ds(start, TB), :],
            sems.at[e]).wait()
    for e in range(E):
        cnt = jnp.sum(maskf[:, e:e + 1]).astype(jnp.int32)
        carry_ref[0, e] = carry_ref[0, e] + ((cnt + 7) // 8) * 8


def _ffn_kernel(bs_ref, xg_ref, W1_ref, b1_ref, W2_ref, b2_ref, yg_ref):
    e = pl.program_id(0)
    j = pl.program_id(1)
    count = bs_ref[NBLK, e]
    nact = (count + TB - 1) // TB

    @pl.when(j * TB < count)
    def _():
        h = jax.nn.relu(
            jnp.dot(xg_ref[0], W1_ref[0], preferred_element_type=jnp.float32)
            + b1_ref[0])
        yg_ref[0] = jnp.dot(h, W2_ref[0], preferred_element_type=jnp.float32)

    @pl.when(j == nact)
    def _():
        yg_ref[0] = jnp.zeros((TB, yg_ref.shape[2]), jnp.float32)


def _combine_kernel(bs_ref, w_ref, lsl_ref, b2_ref, yg_ref, out_ref,
                    win_scr, sems):
    i = pl.program_id(0)
    for e in range(E):
        start = pl.multiple_of(bs_ref[i, e], 8)
        pltpu.make_async_copy(
            yg_ref.at[e, pl.ds(start, TB), :], win_scr.at[e], sems.at[e]
        ).start()
    w = w_ref[...]
    lsl = lsl_ref[...]
    acc = jnp.dot(w, b2_ref[...], preferred_element_type=jnp.float32)
    siota = jax.lax.broadcasted_iota(jnp.int32, (TB, TB), 1)
    for e in range(E):
        start = pl.multiple_of(bs_ref[i, e], 8)
        pltpu.make_async_copy(
            yg_ref.at[e, pl.ds(start, TB), :], win_scr.at[e], sems.at[e]
        ).wait()
        M = jnp.where(lsl[:, e:e + 1] == siota, w[:, e:e + 1], 0.0)
        acc = acc + jnp.dot(M, win_scr[e], preferred_element_type=jnp.float32)
    out_ref[...] = acc


def kernel(x, W1, b1, W2, b2, Wg, bg, Wd1, bd1, Wd2, bd2):
    B, N, Dm = x.shape
    T = B * N
    xf = x.reshape(T, Dm)

    w, lsl, bs, xg = pl.pallas_call(
        _dispatch_kernel,
        grid=(NBLK,),
        in_specs=[
            pl.BlockSpec((TB, Dm), lambda i: (i, 0)),
            pl.BlockSpec((Dm, E), lambda i: (0, 0)),
            pl.BlockSpec((1, E), lambda i: (0, 0)),
            pl.BlockSpec((Dm, Dm // 2), lambda i: (0, 0)),
            pl.BlockSpec((1, Dm // 2), lambda i: (0, 0)),
            pl.BlockSpec((Dm // 2, 1), lambda i: (0, 0)),
            pl.BlockSpec((1, 1), lambda i: (0, 0)),
        ],
        out_specs=[
            pl.BlockSpec((TB, E), lambda i: (i, 0)),
            pl.BlockSpec((TB, E), lambda i: (i, 0)),
            pl.BlockSpec((NBLK + 1, E), lambda i: (0, 0)),
            pl.BlockSpec(memory_space=pltpu.MemorySpace.HBM),
        ],
        out_shape=[
            jax.ShapeDtypeStruct((T, E), jnp.float32),
            jax.ShapeDtypeStruct((T, E), jnp.int32),
            jax.ShapeDtypeStruct((NBLK + 1, E), jnp.int32),
            jax.ShapeDtypeStruct((E, T, Dm), jnp.float32),
        ],
        scratch_shapes=[
            pltpu.VMEM((E, TB, Dm), jnp.float32),
            pltpu.SMEM((1, E), jnp.int32),
            pltpu.SemaphoreType.DMA((E,)),
        ],
        compiler_params=pltpu.CompilerParams(
            dimension_semantics=("arbitrary",)),
    )(xf, Wg, bg.reshape(1, E), Wd1, bd1.reshape(1, Dm // 2),
      Wd2, bd2.reshape(1, 1))

    def _xg_idx(e, j, bs):
        count = bs[NBLK, e]
        nact = (count + TB - 1) // TB
        j_eff = jnp.minimum(j, jnp.minimum(nact, NBLK - 1))
        return (e, j_eff, 0)

    yg = pl.pallas_call(
        _ffn_kernel,
        grid_spec=pltpu.PrefetchScalarGridSpec(
            num_scalar_prefetch=1,
            grid=(E, NBLK),
            in_specs=[
                pl.BlockSpec((1, TB, Dm), _xg_idx),
                pl.BlockSpec((1, Dm, 2 * Dm), lambda e, j, bs: (e, 0, 0)),
                pl.BlockSpec((1, 1, 2 * Dm), lambda e, j, bs: (e, 0, 0)),
                pl.BlockSpec((1, 2 * Dm, Dm), lambda e, j, bs: (e, 0, 0)),
                pl.BlockSpec((1, 1, Dm), lambda e, j, bs: (e, 0, 0)),
            ],
            out_specs=pl.BlockSpec((1, TB, Dm), _xg_idx),
        ),
        out_shape=jax.ShapeDtypeStruct((E, T, Dm), jnp.float32),
        compiler_params=pltpu.CompilerParams(
            dimension_semantics=("arbitrary", "arbitrary")),
    )(bs, xg, W1, b1.reshape(E, 1, 2 * Dm), W2, b2.reshape(E, 1, Dm))

    out = pl.pallas_call(
        _combine_kernel,
        grid_spec=pltpu.PrefetchScalarGridSpec(
            num_scalar_prefetch=1,
            grid=(NBLK,),
            in_specs=[
                pl.BlockSpec((TB, E), lambda i, bs: (i, 0)),
                pl.BlockSpec((TB, E), lambda i, bs: (i, 0)),
                pl.BlockSpec((E, Dm), lambda i, bs: (0, 0)),
                pl.BlockSpec(memory_space=pltpu.MemorySpace.HBM),
            ],
            out_specs=pl.BlockSpec((TB, Dm), lambda i, bs: (i, 0)),
            scratch_shapes=[
                pltpu.VMEM((E, TB, Dm), jnp.float32),
                pltpu.SemaphoreType.DMA((E,)),
            ],
        ),
        out_shape=jax.ShapeDtypeStruct((T, Dm), jnp.float32),
        compiler_params=pltpu.CompilerParams(
            dimension_semantics=("arbitrary",)),
    )(bs, w, lsl, b2, yg)
    return out.reshape(B, N, Dm)


# speedup vs baseline: 2.5948x; 1.1902x over previous
import jax
import jax.numpy as jnp
from jax.experimental import pallas as pl
from jax.experimental.pallas import tpu as pltpu

D = 1024
E = 4
TB = 256
NBLK = 4096 // TB
MIN_K = 1.0
MAX_K = 4.0
TH_LO = 0.5
TH_HI = 2.0


def _routing(xb, Wd1, bd1, Wd2, bd2, Wg, bg):
    hd = jax.nn.relu(jnp.dot(xb, Wd1, preferred_element_type=jnp.float32) + bd1)
    z = jnp.dot(hd, Wd2, preferred_element_type=jnp.float32) + bd2
    ent = jax.nn.softplus(z)
    norm = jnp.clip((ent - TH_LO) / (TH_HI - TH_LO), 0.0, 1.0)
    k = jnp.clip(jnp.round(MIN_K + norm * (MAX_K - MIN_K)), MIN_K, MAX_K)
    logits = jnp.dot(xb, Wg, preferred_element_type=jnp.float32) + bg
    col = jax.lax.broadcasted_iota(jnp.int32, logits.shape, 1)
    rank = jnp.zeros_like(logits)
    for j in range(E):
        lj = logits[:, j:j + 1]
        rank = rank + jnp.where(
            (lj > logits) | ((lj == logits) & (col > j)), 1.0, 0.0)
    mask = rank < k
    m = jnp.max(logits, axis=1, keepdims=True)
    exps = jnp.where(mask, jnp.exp(logits - m), 0.0)
    w = exps / jnp.sum(exps, axis=1, keepdims=True)
    return w, mask


def _dispatch_kernel(x_ref, Wg_ref, bg_ref, Wd1_ref, bd1_ref, Wd2_ref, bd2_ref,
                     w_ref, lsl_ref, bs_ref, xg_ref,
                     gath_scr, carry_ref, sems):
    i = pl.program_id(0)

    @pl.when(i == 0)
    def _():
        for e in range(E):
            carry_ref[0, e] = 0

    xb = x_ref[...]
    w, mask = _routing(xb, Wd1_ref[...], bd1_ref[...], Wd2_ref[...],
                       bd2_ref[...], Wg_ref[...], bg_ref[...])
    maskf = mask.astype(jnp.float32)
    r0 = jax.lax.broadcasted_iota(jnp.int32, (TB, TB), 0)
    r1 = jax.lax.broadcasted_iota(jnp.int32, (TB, TB), 1)
    L = (r0 >= r1).astype(jnp.float32)
    cum = jnp.dot(L, maskf, preferred_element_type=jnp.float32)
    lsl = (cum - 1.0).astype(jnp.int32)

    w_ref[...] = w
    lsl_ref[...] = lsl

    col1 = jax.lax.broadcasted_iota(jnp.int32, (1, E), 1)
    vc = jnp.zeros((1, E), jnp.int32)
    for e in range(E):
        vc = jnp.where(col1 == e, carry_ref[0, e], vc)
    csum = jnp.sum(maskf, axis=0, keepdims=True).astype(jnp.int32)
    vc_new = vc + ((csum + 7) // 8) * 8
    riota = jax.lax.broadcasted_iota(jnp.int32, (NBLK + 1, E), 0)
    bs = jnp.where(riota == i, vc, bs_ref[...])
    bs_ref[...] = jnp.where(riota == NBLK, vc_new, bs)

    siota = jax.lax.broadcasted_iota(jnp.int32, (TB, TB), 1)
    gaths = []
    for e in range(E):
        ohT = jnp.where((lsl[:, e:e + 1] == siota) & mask[:, e:e + 1],
                        1.0, 0.0)
        gaths.append(jax.lax.dot_general(
            ohT, xb, (((0,), (0,)), ((), ())),
            preferred_element_type=jnp.float32))

    @pl.when(i > 0)
    def _():
        for e in range(E):
            start = pl.multiple_of(
                jnp.minimum(carry_ref[0, e], (NBLK - 1) * TB), 8)
            pltpu.make_async_copy(
                gath_scr.at[e], xg_ref.at[e, pl.ds(start, TB), :],
                sems.at[e]).wait()

    for e in range(E):
        gath_scr[e] = gaths[e]
        start = pl.multiple_of(carry_ref[0, e], 8)
        pltpu.make_async_copy(
            gath_scr.at[e], xg_ref.at[e, pl.ds(start, TB), :],
            sems.at[e]).start()
    for e in range(E):
        cnt = jnp.sum(maskf[:, e:e + 1]).astype(jnp.int32)
        carry_ref[0, e] = carry_ref[0, e] + ((cnt + 7) // 8) * 8

    @pl.when(i == NBLK - 1)
    def _():
        for e in range(E):
            start = pl.multiple_of(
                jnp.minimum(carry_ref[0, e], (NBLK - 1) * TB), 8)
            pltpu.make_async_copy(
                gath_scr.at[e], xg_ref.at[e, pl.ds(start, TB), :],
                sems.at[e]).wait()


def _ffn_kernel(bs_ref, xg_ref, W1_ref, b1_ref, W2_ref, b2_ref, yg_ref):
    e = pl.program_id(0)
    j = pl.program_id(1)
    count = bs_ref[NBLK, e]
    nact = (count + TB - 1) // TB

    @pl.when(j * TB < count)
    def _():
        h = jax.nn.relu(
            jnp.dot(xg_ref[0], W1_ref[0], preferred_element_type=jnp.float32)
            + b1_ref[0])
        yg_ref[0] = jnp.dot(h, W2_ref[0], preferred_element_type=jnp.float32)

    @pl.when(j == nact)
    def _():
        yg_ref[0] = jnp.zeros((TB, yg_ref.shape[2]), jnp.float32)


def _combine_kernel(bs_ref, w_ref, lsl_ref, b2_ref, yg_ref, out_ref,
                    win_scr, sems):
    i = pl.program_id(0)
    buf = jax.lax.rem(i, 2)

    def _start(step, b):
        for e in range(E):
            start = pl.multiple_of(bs_ref[step, e], 8)
            pltpu.make_async_copy(
                yg_ref.at[e, pl.ds(start, TB), :], win_scr.at[b, e],
                sems.at[b, e]).start()

    def _wait(step, b):
        for e in range(E):
            start = pl.multiple_of(bs_ref[step, e], 8)
            pltpu.make_async_copy(
                yg_ref.at[e, pl.ds(start, TB), :], win_scr.at[b, e],
                sems.at[b, e]).wait()

    @pl.when(i == 0)
    def _():
        _start(0, 0)

    @pl.when(i < NBLK - 1)
    def _():
        _start(i + 1, 1 - buf)

    w = w_ref[...]
    lsl = lsl_ref[...]
    acc = jnp.dot(w, b2_ref[...], preferred_element_type=jnp.float32)
    siota = jax.lax.broadcasted_iota(jnp.int32, (TB, TB), 1)
    _wait(i, buf)
    for e in range(E):
        M = jnp.where(lsl[:, e:e + 1] == siota, w[:, e:e + 1], 0.0)
        acc = acc + jnp.dot(M, win_scr[buf, e],
                            preferred_element_type=jnp.float32)
    out_ref[...] = acc


def kernel(x, W1, b1, W2, b2, Wg, bg, Wd1, bd1, Wd2, bd2):
    B, N, Dm = x.shape
    T = B * N
    xf = x.reshape(T, Dm)

    w, lsl, bs, xg = pl.pallas_call(
        _dispatch_kernel,
        grid=(NBLK,),
        in_specs=[
            pl.BlockSpec((TB, Dm), lambda i: (i, 0)),
            pl.BlockSpec((Dm, E), lambda i: (0, 0)),
            pl.BlockSpec((1, E), lambda i: (0, 0)),
            pl.BlockSpec((Dm, Dm // 2), lambda i: (0, 0)),
            pl.BlockSpec((1, Dm // 2), lambda i: (0, 0)),
            pl.BlockSpec((Dm // 2, 1), lambda i: (0, 0)),
            pl.BlockSpec((1, 1), lambda i: (0, 0)),
        ],
        out_specs=[
            pl.BlockSpec((TB, E), lambda i: (i, 0)),
            pl.BlockSpec((TB, E), lambda i: (i, 0)),
            pl.BlockSpec((NBLK + 1, E), lambda i: (0, 0)),
            pl.BlockSpec(memory_space=pltpu.MemorySpace.HBM),
        ],
        out_shape=[
            jax.ShapeDtypeStruct((T, E), jnp.float32),
            jax.ShapeDtypeStruct((T, E), jnp.int32),
            jax.ShapeDtypeStruct((NBLK + 1, E), jnp.int32),
            jax.ShapeDtypeStruct((E, T, Dm), jnp.float32),
        ],
        scratch_shapes=[
            pltpu.VMEM((E, TB, Dm), jnp.float32),
            pltpu.SMEM((1, E), jnp.int32),
            pltpu.SemaphoreType.DMA((E,)),
        ],
        compiler_params=pltpu.CompilerParams(
            dimension_semantics=("arbitrary",)),
    )(xf, Wg, bg.reshape(1, E), Wd1, bd1.reshape(1, Dm // 2),
      Wd2, bd2.reshape(1, 1))

    def _xg_idx(e, j, bs):
        count = bs[NBLK, e]
        nact = (count + TB - 1) // TB
        j_eff = jnp.minimum(j, jnp.minimum(nact, NBLK - 1))
        return (e, j_eff, 0)

    yg = pl.pallas_call(
        _ffn_kernel,
        grid_spec=pltpu.PrefetchScalarGridSpec(
            num_scalar_prefetch=1,
            grid=(E, NBLK),
            in_specs=[
                pl.BlockSpec((1, TB, Dm), _xg_idx),
                pl.BlockSpec((1, Dm, 2 * Dm), lambda e, j, bs: (e, 0, 0)),
                pl.BlockSpec((1, 1, 2 * Dm), lambda e, j, bs: (e, 0, 0)),
                pl.BlockSpec((1, 2 * Dm, Dm), lambda e, j, bs: (e, 0, 0)),
                pl.BlockSpec((1, 1, Dm), lambda e, j, bs: (e, 0, 0)),
            ],
            out_specs=pl.BlockSpec((1, TB, Dm), _xg_idx),
        ),
        out_shape=jax.ShapeDtypeStruct((E, T, Dm), jnp.float32),
        compiler_params=pltpu.CompilerParams(
            dimension_semantics=("arbitrary", "arbitrary")),
    )(bs, xg, W1, b1.reshape(E, 1, 2 * Dm), W2, b2.reshape(E, 1, Dm))

    out = pl.pallas_call(
        _combine_kernel,
        grid_spec=pltpu.PrefetchScalarGridSpec(
            num_scalar_prefetch=1,
            grid=(NBLK,),
            in_specs=[
                pl.BlockSpec((TB, E), lambda i, bs: (i, 0)),
                pl.BlockSpec((TB, E), lambda i, bs: (i, 0)),
                pl.BlockSpec((E, Dm), lambda i, bs: (0, 0)),
                pl.BlockSpec(memory_space=pltpu.MemorySpace.HBM),
            ],
            out_specs=pl.BlockSpec((TB, Dm), lambda i, bs: (i, 0)),
            scratch_shapes=[
                pltpu.VMEM((2, E, TB, Dm), jnp.float32),
                pltpu.SemaphoreType.DMA((2, E)),
            ],
        ),
        out_shape=jax.ShapeDtypeStruct((T, Dm), jnp.float32),
        compiler_params=pltpu.CompilerParams(
            dimension_semantics=("arbitrary",)),
    )(bs, w, lsl, b2, yg)
    return out.reshape(B, N, Dm)
